# baseline (speedup 1.0000x reference)
import jax
import jax.numpy as jnp
from jax.experimental import pallas as pl
from jax.experimental.pallas import tpu as pltpu

_B, _K, _N, _T, _M = 4, 4, 1024, 16, 5
_TM = _T * _M
_BT = _B * _T
_BLK = 512


def _spectral_kernel(x_ref, w1_ref, b1_ref, w2_ref, b2_ref, l_ref,
                     out_ref, z_ref, vb_ref):
    i = pl.program_id(0)

    @pl.when(i == 0)
    def _init():
        w1c = w1_ref[...].reshape(_TM, _M, _T).sum(axis=1)
        vt = jax.lax.dot_general(w1c, w2_ref[...],
                                 (((0,), (1,)), ((), ())),
                                 preferred_element_type=jnp.float32)
        z_ref[...] = jnp.concatenate(
            [jnp.dot(x_ref[b, 0], vt, preferred_element_type=jnp.float32)
             for b in range(_B)], axis=1).astype(jnp.bfloat16)
        beff = jax.lax.dot_general(b1_ref[...], w2_ref[...],
                                   (((1,), (1,)), ((), ())),
                                   preferred_element_type=jnp.float32)
        vb_ref[...] = jnp.tile(beff + b2_ref[...], (1, _B))

    lsum = (l_ref[0] + l_ref[1]) + (l_ref[2] + l_ref[3])
    out_ref[...] = jnp.dot(lsum.astype(jnp.bfloat16), z_ref[...],
                           preferred_element_type=jnp.float32) + vb_ref[...]


def kernel(x, mul_L, W1, b1, W2, b2):
    out = pl.pallas_call(
        _spectral_kernel,
        grid=(_N // _BLK,),
        in_specs=[
            pl.BlockSpec((_B, 1, _N, _T), lambda i: (0, 0, 0, 0)),
            pl.BlockSpec((_TM, _TM), lambda i: (0, 0)),
            pl.BlockSpec((1, _TM), lambda i: (0, 0)),
            pl.BlockSpec((_T, _TM), lambda i: (0, 0)),
            pl.BlockSpec((1, _T), lambda i: (0, 0)),
            pl.BlockSpec((_K, _BLK, _N), lambda i: (0, i, 0)),
        ],
        out_specs=pl.BlockSpec((_BLK, _BT), lambda i: (i, 0)),
        out_shape=jax.ShapeDtypeStruct((_N, _BT), jnp.float32),
        scratch_shapes=[pltpu.VMEM((_N, _BT), jnp.bfloat16),
                        pltpu.VMEM((1, _BT), jnp.float32)],
    )(x, W1, b1.reshape(1, _TM), W2, b2.reshape(1, _T), mul_L)
    return out.reshape(_N, _B, _T).transpose(1, 0, 2)


# speedup vs baseline: 1.0735x; 1.0735x over previous
import jax
import jax.numpy as jnp
from jax.experimental import pallas as pl
from jax.experimental.pallas import tpu as pltpu

_B, _K, _N, _T, _M = 4, 4, 1024, 16, 5
_TM = _T * _M
_BT = _B * _T
_CHUNKS = ((0, 384), (384, 384), (768, 192), (960, 64))


def _spectral_kernel(xc_ref, w1_ref, b1_ref, w2_ref, b2_ref, l_hbm,
                     out_ref, lbuf, sem):
    def _copies(c):
        base, rows = _CHUNKS[c]
        return [pltpu.make_async_copy(
                    l_hbm.at[k, pl.ds(base, rows), :],
                    lbuf.at[k, pl.ds(base, rows), :],
                    sem.at[c]) for k in range(_K)]

    def _start(c):
        for cp in _copies(c):
            cp.start()

    _start(0)
    _start(1)

    w1c = w1_ref[...].reshape(_TM, _M, _T).sum(axis=1)
    vt = jax.lax.dot_general(w1c, w2_ref[...],
                             (((0,), (1,)), ((), ())),
                             preferred_element_type=jnp.float32)
    row = jax.lax.broadcasted_iota(jnp.int32, (_BT, _BT), 0) // _T
    col = jax.lax.broadcasted_iota(jnp.int32, (_BT, _BT), 1) // _T
    d = jnp.where(row == col, jnp.tile(vt, (_B, _B)), 0.0)
    z = jnp.dot(xc_ref[...], d,
                preferred_element_type=jnp.float32).astype(jnp.bfloat16)
    beff = jax.lax.dot_general(b1_ref[...], w2_ref[...],
                               (((1,), (1,)), ((), ())),
                               preferred_element_type=jnp.float32)
    vb = jnp.tile(beff + b2_ref[...], (1, _B))

    for c in range(len(_CHUNKS)):
        for cp in _copies(c):
            cp.wait()
        if c + 2 < len(_CHUNKS):
            _start(c + 2)
        base, rows = _CHUNKS[c]
        sl = pl.ds(base, rows)
        lsum = ((lbuf[0, sl, :] + lbuf[1, sl, :])
                + (lbuf[2, sl, :] + lbuf[3, sl, :]))
        out_ref[sl, :] = jnp.dot(
            lsum.astype(jnp.bfloat16), z,
            preferred_element_type=jnp.float32) + vb


def kernel(x, mul_L, W1, b1, W2, b2):
    xc = jnp.transpose(x[:, 0], (1, 0, 2)).reshape(_N, _BT)
    out = pl.pallas_call(
        _spectral_kernel,
        in_specs=[
            pl.BlockSpec((_N, _BT), lambda: (0, 0)),
            pl.BlockSpec((_TM, _TM), lambda: (0, 0)),
            pl.BlockSpec((1, _TM), lambda: (0, 0)),
            pl.BlockSpec((_T, _TM), lambda: (0, 0)),
            pl.BlockSpec((1, _T), lambda: (0, 0)),
            pl.BlockSpec(memory_space=pltpu.HBM),
        ],
        out_specs=pl.BlockSpec((_N, _BT), lambda: (0, 0)),
        out_shape=jax.ShapeDtypeStruct((_N, _BT), jnp.float32),
        scratch_shapes=[pltpu.VMEM((_K, _N, _N), jnp.float32),
                        pltpu.SemaphoreType.DMA((len(_CHUNKS),))],
        compiler_params=pltpu.CompilerParams(
            vmem_limit_bytes=50 * 1024 * 1024),
    )(xc, W1, b1.reshape(1, _TM), W2, b2.reshape(1, _T), mul_L)
    return out.reshape(_N, _B, _T).transpose(1, 0, 2)


# speedup vs baseline: 1.1236x; 1.0466x over previous
import jax
import jax.numpy as jnp
from jax.experimental import pallas as pl
from jax.experimental.pallas import tpu as pltpu

_B, _K, _N, _T, _M = 4, 4, 1024, 16, 5
_TM = _T * _M
_BT = _B * _T
_BLK = 512


def _spectral_kernel(xc_ref, w1_ref, b1_ref, w2_ref, b2_ref, l_ref,
                     out_ref, z_ref, vb_ref):
    i = pl.program_id(0)

    @pl.when(i == 0)
    def _init():
        w1c = w1_ref[...].reshape(_TM, _M, _T).sum(axis=1)
        vt = jax.lax.dot_general(w1c, w2_ref[...],
                                 (((0,), (1,)), ((), ())),
                                 preferred_element_type=jnp.float32)
        row = jax.lax.broadcasted_iota(jnp.int32, (_BT, _BT), 0) // _T
        col = jax.lax.broadcasted_iota(jnp.int32, (_BT, _BT), 1) // _T
        d = jnp.where(row == col, jnp.tile(vt, (_B, _B)), 0.0)
        z_ref[...] = jnp.dot(xc_ref[...], d,
                             preferred_element_type=jnp.float32
                             ).astype(jnp.bfloat16)
        beff = jax.lax.dot_general(b1_ref[...], w2_ref[...],
                                   (((1,), (1,)), ((), ())),
                                   preferred_element_type=jnp.float32)
        vb_ref[...] = jnp.tile(beff + b2_ref[...], (1, _B))

    lsum = (l_ref[0] + l_ref[1]) + (l_ref[2] + l_ref[3])
    out_ref[...] = jnp.dot(lsum.astype(jnp.bfloat16), z_ref[...],
                           preferred_element_type=jnp.float32) + vb_ref[...]


def kernel(x, mul_L, W1, b1, W2, b2):
    xc = jnp.transpose(x[:, 0], (1, 0, 2)).reshape(_N, _BT)
    out = pl.pallas_call(
        _spectral_kernel,
        grid=(_N // _BLK,),
        in_specs=[
            pl.BlockSpec((_N, _BT), lambda i: (0, 0)),
            pl.BlockSpec((_TM, _TM), lambda i: (0, 0)),
            pl.BlockSpec((1, _TM), lambda i: (0, 0)),
            pl.BlockSpec((_T, _TM), lambda i: (0, 0)),
            pl.BlockSpec((1, _T), lambda i: (0, 0)),
            pl.BlockSpec((_K, _BLK, _N), lambda i: (0, i, 0)),
        ],
        out_specs=pl.BlockSpec((_BLK, _BT), lambda i: (i, 0)),
        out_shape=jax.ShapeDtypeStruct((_N, _BT), jnp.float32),
        scratch_shapes=[pltpu.VMEM((_N, _BT), jnp.bfloat16),
                        pltpu.VMEM((1, _BT), jnp.float32)],
    )(xc, W1, b1.reshape(1, _TM), W2, b2.reshape(1, _T), mul_L)
    return out.reshape(_N, _B, _T).transpose(1, 0, 2)
